# baseline (speedup 1.0000x reference)
import functools

import jax
import jax.numpy as jnp
from jax import lax
from jax.experimental import pallas as pl
from jax.experimental.pallas import tpu as pltpu
from jax.experimental.pallas import tpu_sc as plsc

_VOCAB = 100000
_EMBED = 64
_BATCH = 16384
_NEG = 20

_NC = 2
_NS = 16
_NW = _NC * _NS
_BPW = _BATCH // _NW
_CH = 32
_NCH = _BPW // _CH
_NG = _CH // 16
_NQ = _NEG * _CH // 128


def _fire_stage(j, out_embed, in_embed, idxc_v, idxo_v, idxn_v,
                cring, nring, sem):
    pltpu.async_copy(in_embed.at[idxc_v.at[j]], cring.at[0], sem)
    pltpu.async_copy(out_embed.at[idxo_v.at[j]], cring.at[1], sem)
    for q in range(_NQ):
        pltpu.async_copy(out_embed.at[idxn_v.at[j, q]], nring.at[q], sem)


def _drain_stage(dummy32, dummy128, cring, nring, sem):
    for r in range(2):
        pltpu.make_async_copy(dummy32, cring.at[r], sem).wait()
    for q in range(_NQ):
        pltpu.make_async_copy(dummy128, nring.at[q], sem).wait()


def _dots_chunk(cring, nring, dv_pos, dv_neg):
    rc_v = cring.at[0]

    def group_body(g, _):
        rid = lax.iota(jnp.int32, 16) + g * 16
        rids = [rid + (m * 32) for m in range(4)]

        def dc_body(dc, accs):
            accs = list(accs)
            d0 = dc * 16
            for j in range(16):
                col = jnp.broadcast_to(d0 + j, (16,))
                vc = plsc.load_gather(rc_v, [rid, col])
                accs[0] = accs[0] + vc * plsc.load_gather(
                    cring.at[1], [rid, col])
                for k in range(_NEG):
                    accs[1 + k] = accs[1 + k] + vc * plsc.load_gather(
                        nring.at[k // 4], [rids[k % 4], col])
            return tuple(accs)

        accs = lax.fori_loop(0, _EMBED // 16, dc_body,
                             tuple(jnp.zeros((16,), jnp.float32)
                                   for _ in range(_NEG + 1)))
        dv_pos[pl.ds(g * 16, 16)] = accs[0]
        for k in range(_NEG):
            dv_neg[k, pl.ds(g * 16, 16)] = accs[1 + k]
        return 0

    lax.fori_loop(0, _NG, group_body, 0)


def _sc_body(centers3, contexts3, neg4, in_embed, out_embed,
             out_pos, out_neg,
             idxc_v, idxo_v, idxn_v, cring0, cring1, nring0, nring1,
             dv_pos, dv_neg, sem0, sem1):
    wid = lax.axis_index("s") * _NC + lax.axis_index("c")
    base = wid * _BPW
    pltpu.sync_copy(centers3.at[wid], idxc_v)
    pltpu.sync_copy(contexts3.at[wid], idxo_v)
    pltpu.sync_copy(neg4.at[wid], idxn_v)
    _fire_stage(0, out_embed, in_embed, idxc_v, idxo_v, idxn_v,
                cring0, nring0, sem0)
    _fire_stage(1, out_embed, in_embed, idxc_v, idxo_v, idxn_v,
                cring1, nring1, sem1)
    dummy32 = in_embed.at[pl.ds(0, _CH)]
    dummy128 = in_embed.at[pl.ds(0, 128)]

    def chunk_body(j, _):
        b0 = base + j * _CH
        parity = lax.rem(j, 2)

        def do_slot(cring, nring, sem):
            _drain_stage(dummy32, dummy128, cring, nring, sem)
            _dots_chunk(cring, nring, dv_pos, dv_neg)

            @pl.when(j + 2 < _NCH)
            def _():
                _fire_stage(j + 2, out_embed, in_embed,
                            idxc_v, idxo_v, idxn_v, cring, nring, sem)

        @pl.when(parity == 0)
        def _():
            do_slot(cring0, nring0, sem0)

        @pl.when(parity == 1)
        def _():
            do_slot(cring1, nring1, sem1)

        pltpu.sync_copy(dv_pos, out_pos.at[pl.ds(b0, _CH)])
        pltpu.sync_copy(dv_neg, out_neg.at[:, pl.ds(b0, _CH)])
        return 0

    lax.fori_loop(0, _NCH, chunk_body, 0)


_sc_dots = functools.partial(
    pl.kernel,
    out_type=[
        jax.ShapeDtypeStruct((_BATCH,), jnp.float32),
        jax.ShapeDtypeStruct((_NEG, _BATCH), jnp.float32),
    ],
    mesh=plsc.VectorSubcoreMesh(core_axis_name="c", subcore_axis_name="s"),
    scratch_types=[
        pltpu.VMEM((_NCH, _CH), jnp.int32),
        pltpu.VMEM((_NCH, _CH), jnp.int32),
        pltpu.VMEM((_NCH, _NQ, 128), jnp.int32),
        pltpu.VMEM((2, _CH, _EMBED), jnp.float32),
        pltpu.VMEM((2, _CH, _EMBED), jnp.float32),
        pltpu.VMEM((_NQ, 128, _EMBED), jnp.float32),
        pltpu.VMEM((_NQ, 128, _EMBED), jnp.float32),
        pltpu.VMEM((_CH,), jnp.float32),
        pltpu.VMEM((_NEG, _CH), jnp.float32),
        pltpu.SemaphoreType.DMA,
        pltpu.SemaphoreType.DMA,
    ],
    compiler_params=pltpu.CompilerParams(use_tc_tiling_on_sc=False,
                                         needs_layout_passes=False),
)(_sc_body)


_TC_BB = 2048


def _log_sigmoid(x):
    return jnp.minimum(x, 0.0) - jnp.log1p(jnp.exp(-jnp.abs(x)))


def _tc_loss_body(pos_ref, neg_ref, out_ref):
    i = pl.program_id(0)
    part = (jnp.sum(_log_sigmoid(pos_ref[...]))
            + jnp.sum(_log_sigmoid(-neg_ref[...])))

    @pl.when(i == 0)
    def _():
        out_ref[0, 0] = 0.0

    out_ref[0, 0] += part

    @pl.when(i == pl.num_programs(0) - 1)
    def _():
        out_ref[0, 0] = out_ref[0, 0] * (-1.0 / _BATCH)


_tc_loss = pl.pallas_call(
    _tc_loss_body,
    grid=(_BATCH // _TC_BB,),
    in_specs=[
        pl.BlockSpec((_TC_BB,), lambda i: (i,)),
        pl.BlockSpec((_NEG, _TC_BB), lambda i: (0, i)),
    ],
    out_specs=pl.BlockSpec(memory_space=pltpu.SMEM),
    out_shape=jax.ShapeDtypeStruct((1, 1), jnp.float32),
)


def kernel(centers, contexts, negatives, in_embed, out_embed):
    centers3 = centers.astype(jnp.int32).reshape(_NW, _NCH, _CH)
    contexts3 = contexts.astype(jnp.int32).reshape(_NW, _NCH, _CH)
    neg4 = (negatives.astype(jnp.int32)
            .reshape(_NW, _NCH, _CH, _NEG)
            .transpose(0, 1, 3, 2)
            .reshape(_NW, _NCH, _NQ, 128))
    pos_dot, neg_dot = _sc_dots(centers3, contexts3, neg4,
                                in_embed, out_embed)
    loss = _tc_loss(pos_dot, neg_dot)
    return loss[0, 0]


# speedup vs baseline: 2.8573x; 2.8573x over previous
import functools

import jax
import jax.numpy as jnp
from jax import lax
from jax.experimental import pallas as pl
from jax.experimental.pallas import tpu as pltpu
from jax.experimental.pallas import tpu_sc as plsc

_VOCAB = 100000
_EMBED = 64
_BATCH = 16384
_NEG = 20

_NC = 2
_NS = 16
_NW = _NC * _NS
_BPW = _BATCH // _NW
_CH = 32
_NCH = _BPW // _CH
_NG = _CH // 16
_NB = _NEG + 2


def _fire_stage(j, out_embed, in_embed, idxc_v, idxo_v, idxn_v, ring, sem):
    pltpu.async_copy(in_embed.at[idxc_v.at[j]], ring.at[0], sem)
    pltpu.async_copy(out_embed.at[idxo_v.at[j]], ring.at[1], sem)
    for k in range(_NEG):
        pltpu.async_copy(out_embed.at[idxn_v.at[j, k]], ring.at[2 + k], sem)


def _drain_stage(dummy_hbm, ring, sem):
    for r in range(_NB):
        pltpu.make_async_copy(dummy_hbm, ring.at[r], sem).wait()


def _dots_chunk(ring, dv_pos, dv_neg):
    rc_v = ring.at[0]
    bufs = [ring.at[1 + r] for r in range(_NEG + 1)]

    def group_body(g, _):
        lane = lax.iota(jnp.int32, 16)
        rid = lane + g * 16

        def dc_body(dc, accs):
            accs = list(accs)
            d0 = dc * 16
            for j in range(16):
                col = (jnp.broadcast_to(d0 + j, (16,)) + lane) & 63
                vc = plsc.load_gather(rc_v, [rid, col])
                for r in range(len(bufs)):
                    accs[r] = accs[r] + vc * plsc.load_gather(bufs[r], [rid, col])
            return tuple(accs)

        accs = lax.fori_loop(0, _EMBED // 16, dc_body,
                             tuple(jnp.zeros((16,), jnp.float32)
                                   for _ in range(len(bufs))))
        dv_pos[pl.ds(g * 16, 16)] = accs[0]
        for k in range(_NEG):
            dv_neg[k, pl.ds(g * 16, 16)] = accs[1 + k]
        return 0

    lax.fori_loop(0, _NG, group_body, 0)


def _sc_body(centers3, contexts3, neg4, in_embed, out_embed,
             out_pos, out_neg,
             idxc_v, idxo_v, idxn_v, ring0, ring1, dv_pos, dv_neg,
             sem0, sem1):
    wid = lax.axis_index("s") * _NC + lax.axis_index("c")
    base = wid * _BPW
    pltpu.sync_copy(centers3.at[wid], idxc_v)
    pltpu.sync_copy(contexts3.at[wid], idxo_v)
    pltpu.sync_copy(neg4.at[wid], idxn_v)
    _fire_stage(0, out_embed, in_embed, idxc_v, idxo_v, idxn_v, ring0, sem0)
    _fire_stage(1, out_embed, in_embed, idxc_v, idxo_v, idxn_v, ring1, sem1)
    dummy = in_embed.at[pl.ds(0, _CH)]

    def chunk_body(j, _):
        b0 = base + j * _CH
        parity = lax.rem(j, 2)

        def do_slot(ring, sem):
            _drain_stage(dummy, ring, sem)
            _dots_chunk(ring, dv_pos, dv_neg)

            @pl.when(j + 2 < _NCH)
            def _():
                _fire_stage(j + 2, out_embed, in_embed,
                            idxc_v, idxo_v, idxn_v, ring, sem)

        @pl.when(parity == 0)
        def _():
            do_slot(ring0, sem0)

        @pl.when(parity == 1)
        def _():
            do_slot(ring1, sem1)

        pltpu.sync_copy(dv_pos, out_pos.at[pl.ds(b0, _CH)])
        pltpu.sync_copy(dv_neg, out_neg.at[:, pl.ds(b0, _CH)])
        return 0

    lax.fori_loop(0, _NCH, chunk_body, 0)


_sc_dots = functools.partial(
    pl.kernel,
    out_type=[
        jax.ShapeDtypeStruct((_BATCH,), jnp.float32),
        jax.ShapeDtypeStruct((_NEG, _BATCH), jnp.float32),
    ],
    mesh=plsc.VectorSubcoreMesh(core_axis_name="c", subcore_axis_name="s"),
    scratch_types=[
        pltpu.VMEM((_NCH, _CH), jnp.int32),
        pltpu.VMEM((_NCH, _CH), jnp.int32),
        pltpu.VMEM((_NCH, _NEG, _CH), jnp.int32),
        pltpu.VMEM((_NB, _CH, _EMBED), jnp.float32),
        pltpu.VMEM((_NB, _CH, _EMBED), jnp.float32),
        pltpu.VMEM((_CH,), jnp.float32),
        pltpu.VMEM((_NEG, _CH), jnp.float32),
        pltpu.SemaphoreType.DMA,
        pltpu.SemaphoreType.DMA,
    ],
    compiler_params=pltpu.CompilerParams(use_tc_tiling_on_sc=False,
                                         needs_layout_passes=False),
)(_sc_body)


_TC_BB = 2048


def _log_sigmoid(x):
    return jnp.minimum(x, 0.0) - jnp.log1p(jnp.exp(-jnp.abs(x)))


def _tc_loss_body(pos_ref, neg_ref, out_ref):
    i = pl.program_id(0)
    part = (jnp.sum(_log_sigmoid(pos_ref[...]))
            + jnp.sum(_log_sigmoid(-neg_ref[...])))

    @pl.when(i == 0)
    def _():
        out_ref[0, 0] = 0.0

    out_ref[0, 0] += part

    @pl.when(i == pl.num_programs(0) - 1)
    def _():
        out_ref[0, 0] = out_ref[0, 0] * (-1.0 / _BATCH)


_tc_loss = pl.pallas_call(
    _tc_loss_body,
    grid=(_BATCH // _TC_BB,),
    in_specs=[
        pl.BlockSpec((_TC_BB,), lambda i: (i,)),
        pl.BlockSpec((_NEG, _TC_BB), lambda i: (0, i)),
    ],
    out_specs=pl.BlockSpec(memory_space=pltpu.SMEM),
    out_shape=jax.ShapeDtypeStruct((1, 1), jnp.float32),
)


def kernel(centers, contexts, negatives, in_embed, out_embed):
    centers3 = centers.astype(jnp.int32).reshape(_NW, _NCH, _CH)
    contexts3 = contexts.astype(jnp.int32).reshape(_NW, _NCH, _CH)
    neg4 = (negatives.astype(jnp.int32)
            .reshape(_NW, _NCH, _CH, _NEG)
            .transpose(0, 1, 3, 2))
    pos_dot, neg_dot = _sc_dots(centers3, contexts3, neg4,
                                in_embed, out_embed)
    loss = _tc_loss(pos_dot, neg_dot)
    return loss[0, 0]


# speedup vs baseline: 2.9546x; 1.0340x over previous
import functools

import jax
import jax.numpy as jnp
from jax import lax
from jax.experimental import pallas as pl
from jax.experimental.pallas import tpu as pltpu
from jax.experimental.pallas import tpu_sc as plsc

_VOCAB = 100000
_EMBED = 64
_BATCH = 16384
_NEG = 20

_NC = 2
_NS = 16
_NW = _NC * _NS
_BPW = _BATCH // _NW
_CH = 32
_NCH = _BPW // _CH
_NG = _CH // 16
_NB = _NEG + 2


_NQ = _NEG * _CH // 128


def _fire_stage(j, out_embed, in_embed, idxc_v, idxo_v, idxn_v,
                cring, nring, sem):
    pltpu.async_copy(in_embed.at[idxc_v.at[j]], cring.at[0], sem)
    pltpu.async_copy(out_embed.at[idxo_v.at[j]], cring.at[1], sem)
    for q in range(_NQ):
        pltpu.async_copy(out_embed.at[idxn_v.at[j, q]],
                         nring.at[pl.ds(q * 128, 128)], sem)


def _drain_stage(dummy32, dummy128, cring, nring, sem):
    for r in range(2):
        pltpu.make_async_copy(dummy32, cring.at[r], sem).wait()
    for q in range(_NQ):
        pltpu.make_async_copy(dummy128, nring.at[pl.ds(q * 128, 128)],
                              sem).wait()


def _dots_chunk(cring, nring, dv_pos, dv_neg):
    rc_v = cring.at[0]
    ro_v = cring.at[1]

    def group_body(g, _):
        lane = lax.iota(jnp.int32, 16)
        rid = lane + g * 16
        rid20 = rid * _NEG

        def dc_body(dc, accs):
            accs = list(accs)
            d0 = dc * 16
            for j in range(16):
                col = (jnp.broadcast_to(d0 + j, (16,)) + lane) & 63
                vc = plsc.load_gather(rc_v, [rid, col])
                accs[0] = accs[0] + vc * plsc.load_gather(ro_v, [rid, col])
                for k in range(_NEG):
                    accs[1 + k] = accs[1 + k] + vc * plsc.load_gather(
                        nring, [rid20 + k, col])
            return tuple(accs)

        accs = lax.fori_loop(0, _EMBED // 16, dc_body,
                             tuple(jnp.zeros((16,), jnp.float32)
                                   for _ in range(_NEG + 1)))
        dv_pos[pl.ds(g * 16, 16)] = accs[0]
        for k in range(_NEG):
            dv_neg[k, pl.ds(g * 16, 16)] = accs[1 + k]
        return 0

    lax.fori_loop(0, _NG, group_body, 0)


def _sc_body(centers3, contexts3, neg4, in_embed, out_embed,
             out_pos, out_neg,
             idxc_v, idxo_v, idxn_v, cring0, cring1, nring0, nring1,
             dv_pos, dv_neg, sem0, sem1):
    wid = lax.axis_index("s") * _NC + lax.axis_index("c")
    base = wid * _BPW
    pltpu.sync_copy(centers3.at[wid], idxc_v)
    pltpu.sync_copy(contexts3.at[wid], idxo_v)
    pltpu.sync_copy(neg4.at[wid], idxn_v)
    _fire_stage(0, out_embed, in_embed, idxc_v, idxo_v, idxn_v,
                cring0, nring0, sem0)
    _fire_stage(1, out_embed, in_embed, idxc_v, idxo_v, idxn_v,
                cring1, nring1, sem1)
    dummy32 = in_embed.at[pl.ds(0, _CH)]
    dummy128 = in_embed.at[pl.ds(0, 128)]

    def chunk_body(j, _):
        b0 = base + j * _CH
        parity = lax.rem(j, 2)

        def do_slot(cring, nring, sem):
            _drain_stage(dummy32, dummy128, cring, nring, sem)
            _dots_chunk(cring, nring, dv_pos, dv_neg)

            @pl.when(j + 2 < _NCH)
            def _():
                _fire_stage(j + 2, out_embed, in_embed,
                            idxc_v, idxo_v, idxn_v, cring, nring, sem)

        @pl.when(parity == 0)
        def _():
            do_slot(cring0, nring0, sem0)

        @pl.when(parity == 1)
        def _():
            do_slot(cring1, nring1, sem1)

        pltpu.sync_copy(dv_pos, out_pos.at[pl.ds(b0, _CH)])
        pltpu.sync_copy(dv_neg, out_neg.at[:, pl.ds(b0, _CH)])
        return 0

    lax.fori_loop(0, _NCH, chunk_body, 0)


_sc_dots = functools.partial(
    pl.kernel,
    out_type=[
        jax.ShapeDtypeStruct((_BATCH,), jnp.float32),
        jax.ShapeDtypeStruct((_NEG, _BATCH), jnp.float32),
    ],
    mesh=plsc.VectorSubcoreMesh(core_axis_name="c", subcore_axis_name="s"),
    scratch_types=[
        pltpu.VMEM((_NCH, _CH), jnp.int32),
        pltpu.VMEM((_NCH, _CH), jnp.int32),
        pltpu.VMEM((_NCH, _NQ, 128), jnp.int32),
        pltpu.VMEM((2, _CH, _EMBED), jnp.float32),
        pltpu.VMEM((2, _CH, _EMBED), jnp.float32),
        pltpu.VMEM((_CH * _NEG, _EMBED), jnp.float32),
        pltpu.VMEM((_CH * _NEG, _EMBED), jnp.float32),
        pltpu.VMEM((_CH,), jnp.float32),
        pltpu.VMEM((_NEG, _CH), jnp.float32),
        pltpu.SemaphoreType.DMA,
        pltpu.SemaphoreType.DMA,
    ],
    compiler_params=pltpu.CompilerParams(use_tc_tiling_on_sc=False,
                                         needs_layout_passes=False),
)(_sc_body)


_TC_BB = 2048


def _log_sigmoid(x):
    return jnp.minimum(x, 0.0) - jnp.log1p(jnp.exp(-jnp.abs(x)))


def _tc_loss_body(pos_ref, neg_ref, out_ref):
    i = pl.program_id(0)
    part = (jnp.sum(_log_sigmoid(pos_ref[...]))
            + jnp.sum(_log_sigmoid(-neg_ref[...])))

    @pl.when(i == 0)
    def _():
        out_ref[0, 0] = 0.0

    out_ref[0, 0] += part

    @pl.when(i == pl.num_programs(0) - 1)
    def _():
        out_ref[0, 0] = out_ref[0, 0] * (-1.0 / _BATCH)


_tc_loss = pl.pallas_call(
    _tc_loss_body,
    grid=(_BATCH // _TC_BB,),
    in_specs=[
        pl.BlockSpec((_TC_BB,), lambda i: (i,)),
        pl.BlockSpec((_NEG, _TC_BB), lambda i: (0, i)),
    ],
    out_specs=pl.BlockSpec(memory_space=pltpu.SMEM),
    out_shape=jax.ShapeDtypeStruct((1, 1), jnp.float32),
)


def kernel(centers, contexts, negatives, in_embed, out_embed):
    centers3 = centers.astype(jnp.int32).reshape(_NW, _NCH, _CH)
    contexts3 = contexts.astype(jnp.int32).reshape(_NW, _NCH, _CH)
    neg4 = negatives.astype(jnp.int32).reshape(_NW, _NCH, _NQ, 128)
    pos_dot, neg_dot = _sc_dots(centers3, contexts3, neg4,
                                in_embed, out_embed)
    loss = _tc_loss(pos_dot, neg_dot)
    return loss[0, 0]


# speedup vs baseline: 2.9667x; 1.0041x over previous
import functools

import jax
import jax.numpy as jnp
from jax import lax
from jax.experimental import pallas as pl
from jax.experimental.pallas import tpu as pltpu
from jax.experimental.pallas import tpu_sc as plsc

_VOCAB = 100000
_EMBED = 64
_BATCH = 16384
_NEG = 20

_NC = 2
_NS = 16
_NW = _NC * _NS
_BPW = _BATCH // _NW
_CH = 32
_NCH = _BPW // _CH
_NG = _CH // 16
_NB = _NEG + 2


_NQ = _NEG * _CH // 128


def _fire_stage(j, out_embed, in_embed, idxc_v, idxo_v, idxn_v,
                cring, nring, sem):
    pltpu.async_copy(in_embed.at[idxc_v.at[j]], cring.at[0], sem)
    pltpu.async_copy(out_embed.at[idxo_v.at[j]], cring.at[1], sem)
    for q in range(_NQ):
        pltpu.async_copy(out_embed.at[idxn_v.at[j, q]],
                         nring.at[pl.ds(q * 128, 128)], sem)


def _drain_stage(dummy32, dummy128, cring, nring, sem):
    for r in range(2):
        pltpu.make_async_copy(dummy32, cring.at[r], sem).wait()
    for q in range(_NQ):
        pltpu.make_async_copy(dummy128, nring.at[pl.ds(q * 128, 128)],
                              sem).wait()


def _dots_chunk(cring, nring, dv):
    rc_v = cring.at[0]
    ro_v = cring.at[1]

    def group_body(g, _):
        lane = lax.iota(jnp.int32, 16)
        rid = lane + g * 16
        rid20 = rid * _NEG

        def dc_body(dc, accs):
            accs = list(accs)
            d0 = dc * 16
            for j in range(16):
                col = (jnp.broadcast_to(d0 + j, (16,)) + lane) & 63
                vc = plsc.load_gather(rc_v, [rid, col])
                accs[0] = accs[0] + vc * plsc.load_gather(ro_v, [rid, col])
                for k in range(_NEG):
                    accs[1 + k] = accs[1 + k] + vc * plsc.load_gather(
                        nring, [rid20 + k, col])
            return tuple(accs)

        accs = lax.fori_loop(0, _EMBED // 16, dc_body,
                             tuple(jnp.zeros((16,), jnp.float32)
                                   for _ in range(_NEG + 1)))
        for r in range(_NEG + 1):
            dv[r, pl.ds(g * 16, 16)] = accs[r]
        return 0

    lax.fori_loop(0, _NG, group_body, 0)


def _sc_body(centers3, contexts3, neg4, in_embed, out_embed,
             out_dots,
             idxc_v, idxo_v, idxn_v, cring0, cring1, nring0, nring1,
             dv, sem0, sem1):
    wid = lax.axis_index("s") * _NC + lax.axis_index("c")
    base = wid * _BPW
    pltpu.sync_copy(centers3.at[wid], idxc_v)
    pltpu.sync_copy(contexts3.at[wid], idxo_v)
    pltpu.sync_copy(neg4.at[wid], idxn_v)
    _fire_stage(0, out_embed, in_embed, idxc_v, idxo_v, idxn_v,
                cring0, nring0, sem0)
    _fire_stage(1, out_embed, in_embed, idxc_v, idxo_v, idxn_v,
                cring1, nring1, sem1)
    dummy32 = in_embed.at[pl.ds(0, _CH)]
    dummy128 = in_embed.at[pl.ds(0, 128)]

    def chunk_body(j, _):
        b0 = base + j * _CH
        parity = lax.rem(j, 2)

        def do_slot(cring, nring, sem):
            _drain_stage(dummy32, dummy128, cring, nring, sem)
            _dots_chunk(cring, nring, dv)

            @pl.when(j + 2 < _NCH)
            def _():
                _fire_stage(j + 2, out_embed, in_embed,
                            idxc_v, idxo_v, idxn_v, cring, nring, sem)

        @pl.when(parity == 0)
        def _():
            do_slot(cring0, nring0, sem0)

        @pl.when(parity == 1)
        def _():
            do_slot(cring1, nring1, sem1)

        pltpu.sync_copy(dv, out_dots.at[:, pl.ds(b0, _CH)])
        return 0

    lax.fori_loop(0, _NCH, chunk_body, 0)


_sc_dots = functools.partial(
    pl.kernel,
    out_type=jax.ShapeDtypeStruct((_NEG + 1, _BATCH), jnp.float32),
    mesh=plsc.VectorSubcoreMesh(core_axis_name="c", subcore_axis_name="s"),
    scratch_types=[
        pltpu.VMEM((_NCH, _CH), jnp.int32),
        pltpu.VMEM((_NCH, _CH), jnp.int32),
        pltpu.VMEM((_NCH, _NQ, 128), jnp.int32),
        pltpu.VMEM((2, _CH, _EMBED), jnp.float32),
        pltpu.VMEM((2, _CH, _EMBED), jnp.float32),
        pltpu.VMEM((_CH * _NEG, _EMBED), jnp.float32),
        pltpu.VMEM((_CH * _NEG, _EMBED), jnp.float32),
        pltpu.VMEM((_NEG + 1, _CH), jnp.float32),
        pltpu.SemaphoreType.DMA,
        pltpu.SemaphoreType.DMA,
    ],
    compiler_params=pltpu.CompilerParams(use_tc_tiling_on_sc=False,
                                         needs_layout_passes=False),
)(_sc_body)


_TC_BB = 2048


def _log_sigmoid(x):
    return jnp.minimum(x, 0.0) - jnp.log1p(jnp.exp(-jnp.abs(x)))


def _tc_loss_body(dots_ref, out_ref):
    i = pl.program_id(0)
    x = dots_ref[...]
    part = (jnp.sum(_log_sigmoid(x[0:1, :]))
            + jnp.sum(_log_sigmoid(-x[1:, :])))

    @pl.when(i == 0)
    def _():
        out_ref[0, 0] = 0.0

    out_ref[0, 0] += part

    @pl.when(i == pl.num_programs(0) - 1)
    def _():
        out_ref[0, 0] = out_ref[0, 0] * (-1.0 / _BATCH)


_tc_loss = pl.pallas_call(
    _tc_loss_body,
    grid=(_BATCH // _TC_BB,),
    in_specs=[
        pl.BlockSpec((_NEG + 1, _TC_BB), lambda i: (0, i)),
    ],
    out_specs=pl.BlockSpec(memory_space=pltpu.SMEM),
    out_shape=jax.ShapeDtypeStruct((1, 1), jnp.float32),
)


def kernel(centers, contexts, negatives, in_embed, out_embed):
    centers3 = centers.astype(jnp.int32).reshape(_NW, _NCH, _CH)
    contexts3 = contexts.astype(jnp.int32).reshape(_NW, _NCH, _CH)
    neg4 = negatives.astype(jnp.int32).reshape(_NW, _NCH, _NQ, 128)
    dots = _sc_dots(centers3, contexts3, neg4, in_embed, out_embed)
    loss = _tc_loss(dots)
    return loss[0, 0]
